# Initial kernel scaffold; baseline (speedup 1.0000x reference)
import jax
import jax.numpy as jnp
from jax.experimental import pallas as pl


def kernel(stream, W_q, W_k, W_v, pope_delta, q_router_w, kv_router_w, lora_A_q, lora_B_q, lora_A_k, lora_B_k, lora_A_v, lora_B_v, alpha_up, alpha_down, beta_up, beta_down, smhc_norm_w, smhc_phi_pre, smhc_phi_post, smhc_phi_res, W_pre, W_o, W_pg1, W_pg2):
    raise NotImplementedError("write your pallas kernel here")


# speedup vs baseline: 63.3216x; 63.3216x over previous
import numpy as np
import jax
import jax.numpy as jnp
from jax.experimental import pallas as pl
from jax.experimental.pallas import tpu as pltpu

D_IN = 1024; DK = 16; DV = 16; H = 8; EQ = 32; EK = 24
DKP = 2 * DK; R = max(DK // 4, 1); D_ALPHA = int(DK * 1.618)
CONCAT = H * DV; D_PG = max(int(CONCAT * 0.618), 1)

TB1 = 256
TB2 = 256
TB3 = 256
LSC = 128
TB4 = 256

F32 = jnp.float32


def _k1_body(stream_ref, wq_ref, wk_ref, wv_ref, qr_ref, kr_ref,
             x_ref, qb_ref, kb_ref, vb_ref,
             qidx_ref, qgate_ref, kidx_ref, kgate_ref):
    s = stream_ref[0]
    x = jnp.mean(s, axis=0)
    x_ref[...] = x
    qb_ref[...] = jnp.dot(x, wq_ref[...], preferred_element_type=F32)
    kb_ref[...] = jnp.dot(x, wk_ref[...], preferred_element_type=F32)
    vb_ref[...] = jnp.dot(x, wv_ref[...], preferred_element_type=F32)
    ql = jnp.dot(x, qr_ref[...], preferred_element_type=F32)
    kl = jnp.dot(x, kr_ref[...], preferred_element_type=F32)

    def top1(logits, E):
        idxs, gates = [], []
        for h in range(H):
            seg = logits[:, h * E:(h + 1) * E]
            m = jnp.max(seg, axis=-1, keepdims=True)
            lane = jax.lax.broadcasted_iota(jnp.int32, seg.shape, 1)
            idx = jnp.min(jnp.where(seg == m, lane, E), axis=-1, keepdims=True)
            gate = 1.0 / jnp.sum(jnp.exp(seg - m), axis=-1, keepdims=True)
            idxs.append(idx)
            gates.append(gate)
        return jnp.concatenate(idxs, axis=1), jnp.concatenate(gates, axis=1)

    qi, qg = top1(ql, EQ)
    ki, kg = top1(kl, EK)
    qidx_ref[...] = qi
    qgate_ref[...] = qg
    kidx_ref[...] = ki
    kgate_ref[...] = kg


def _col(ref, h, fill):
    lane = jax.lax.broadcasted_iota(jnp.int32, ref.shape, 1)
    return jnp.sum(jnp.where(lane == h, ref[...], fill), axis=1, keepdims=True)


def _fold_mat(EK_lanes, K):
    r = jax.lax.broadcasted_iota(jnp.int32, (EK_lanes, K), 0)
    c = jax.lax.broadcasted_iota(jnp.int32, (EK_lanes, K), 1)
    return (r % K == c).astype(F32)


def _sel(dense_all, idx, K):
    lane = jax.lax.broadcasted_iota(jnp.int32, dense_all.shape, 1)
    masked = jnp.where(lane // K == idx, dense_all, 0.0)
    return jnp.dot(masked, _fold_mat(dense_all.shape[1], K),
                   preferred_element_type=F32)


def _k2a_body(x_ref, qb_ref, kb_ref, vb_ref,
              qidx_ref, qgate_ref, kidx_ref, kgate_ref,
              cq_ref, sq_ref, ck_ref, sk_ref,
              aq_ref, bq_ref, ak_ref, bk_ref, av_ref, bv_ref,
              qp_ref, kp_ref, vh_ref):
    h = pl.program_id(0)
    x = x_ref[...]
    qi = _col(qidx_ref, h, 0)
    qg = _col(qgate_ref, h, 0.0)
    ki = _col(kidx_ref, h, 0)
    kg = _col(kgate_ref, h, 0.0)

    hq_all = jnp.dot(x, aq_ref[0], preferred_element_type=F32)
    dq_all = jnp.dot(hq_all, bq_ref[0], preferred_element_type=F32)
    dq = _sel(dq_all, qi, DK)
    qh = (qb_ref[...] + dq) * qg * (qi != 0).astype(F32)

    hk_all = jnp.dot(x, ak_ref[0], preferred_element_type=F32)
    dk_all = jnp.dot(hk_all, bk_ref[0], preferred_element_type=F32)
    kh = (kb_ref[...] + _sel(dk_all, ki, DK)) * kg

    hv_all = jnp.dot(x, av_ref[0], preferred_element_type=F32)
    dv_all = jnp.dot(hv_all, bv_ref[0], preferred_element_type=F32)
    vh_ref[0] = (vb_ref[...] + _sel(dv_all, ki, DV)) * kg

    mu_q = jax.nn.softplus(qh)
    qp_ref[0] = jnp.concatenate([mu_q * cq_ref[...], mu_q * sq_ref[...]], axis=-1)
    mu_k = jax.nn.softplus(kh)
    kp_ref[0] = jnp.concatenate([mu_k * ck_ref[...], mu_k * sk_ref[...]], axis=-1)


def _k2b_body(x_ref, kidx_ref, au_ref, ad_ref, bu_ref, bd_ref,
              ax_ref, bx_ref):
    h = pl.program_id(0)
    x = x_ref[...]
    ki = _col(kidx_ref, h, 0)
    ha = jnp.dot(x, au_ref[0], preferred_element_type=F32)
    ha = ha * jax.nn.sigmoid(ha)
    a_all = jnp.dot(ha, ad_ref[0], preferred_element_type=F32)
    ax_ref[0] = 2.0 * jax.nn.sigmoid(_sel(a_all, ki, DKP))
    hb = jnp.dot(x, bu_ref[0], preferred_element_type=F32)
    hb = hb * jax.nn.sigmoid(hb)
    b_all = jnp.dot(hb, bd_ref[0], preferred_element_type=F32)
    lane = jax.lax.broadcasted_iota(jnp.int32, b_all.shape, 1)
    bpre = jnp.sum(jnp.where(lane == ki, b_all, 0.0), axis=1, keepdims=True)
    bx_ref[0] = jnp.broadcast_to(jax.nn.sigmoid(bpre), bx_ref.shape[1:])


def _k3_body(qx_ref, kx_ref, ax_ref, bx_ref, vx_ref, out_ref, u_ref,
             qe_s, ke_s, kbe_s, ae_s):
    @pl.when(pl.program_id(0) == 0)
    def _():
        u_ref[...] = jnp.zeros_like(u_ref)

    L = qx_ref.shape[0]
    lane = jax.lax.broadcasted_iota(jnp.int32, (H, H * DV), 1)
    row = jax.lax.broadcasted_iota(jnp.int32, (H, H * DV), 0)
    exp_m = (lane // DV == row).astype(F32)

    def expand(ref3):
        a = ref3[...].reshape(L * DKP, H)
        return jnp.dot(a, exp_m, preferred_element_type=F32).reshape(L, DKP, H * DV)

    qe_s[...] = expand(qx_ref)
    ke = expand(kx_ref)
    ke_s[...] = ke
    ae_s[...] = expand(ax_ref)
    b_exp = jnp.dot(bx_ref[...], exp_m, preferred_element_type=F32)
    kbe_s[...] = ke * b_exp[:, None, :]

    def body(i, U):
        a_t = ae_s[i]
        k_t = ke_s[i]
        kb_t = kbe_s[i]
        q_t = qe_s[i]
        v_t = vx_ref[pl.ds(i, 1), :]
        Z = a_t * U
        r = jnp.sum(k_t * Z, axis=0, keepdims=True)
        s_new = Z - kb_t * (r - v_t)
        out_ref[pl.ds(i, 1), :] = jnp.sum(q_t * s_new, axis=0, keepdims=True)
        return s_new - U

    u_ref[...] = jax.lax.fori_loop(0, L, body, u_ref[...])


def _k4_body(x_ref, cc_ref, wpre_ref, wo_ref, wpg1_ref, wpg2_ref, y_ref):
    x = x_ref[...]
    pre = jax.nn.sigmoid(jnp.dot(x, wpre_ref[...], preferred_element_type=F32))
    o = jnp.dot(cc_ref[...] * pre, wo_ref[...], preferred_element_type=F32)
    g1 = jnp.dot(x, wpg1_ref[...], preferred_element_type=F32)
    pg = jax.nn.sigmoid(jnp.dot(g1 * jax.nn.sigmoid(g1), wpg2_ref[...],
                                preferred_element_type=F32))
    y_ref[...] = o * pg


def kernel(stream, W_q, W_k, W_v, pope_delta, q_router_w, kv_router_w,
           lora_A_q, lora_B_q, lora_A_k, lora_B_k, lora_A_v, lora_B_v,
           alpha_up, alpha_down, beta_up, beta_down,
           smhc_norm_w, smhc_phi_pre, smhc_phi_post, smhc_phi_res,
           W_pre, W_o, W_pg1, W_pg2):
    Bb, NM, T, D = stream.shape
    dt = stream.dtype

    qr_cat = q_router_w.transpose(1, 0, 2).reshape(D, H * EQ)
    kr_cat = kv_router_w.transpose(1, 0, 2).reshape(D, H * EK)
    aq3 = lora_A_q.reshape(H, EQ, D, R).transpose(0, 2, 1, 3).reshape(H, D, EQ * R)
    ak3 = lora_A_k.reshape(H, EK, D, R).transpose(0, 2, 1, 3).reshape(H, D, EK * R)
    av3 = lora_A_v.reshape(H, EK, D, R).transpose(0, 2, 1, 3).reshape(H, D, EK * R)
    eyeq = jnp.eye(EQ, dtype=dt)
    eyek = jnp.eye(EK, dtype=dt)
    bq_bd = jnp.einsum('herk,ef->herfk', lora_B_q.reshape(H, EQ, R, DK),
                       eyeq).reshape(H, EQ * R, EQ * DK)
    bk_bd = jnp.einsum('herk,ef->herfk', lora_B_k.reshape(H, EK, R, DK),
                       eyek).reshape(H, EK * R, EK * DK)
    bv_bd = jnp.einsum('herk,ef->herfk', lora_B_v.reshape(H, EK, R, DV),
                       eyek).reshape(H, EK * R, EK * DV)
    au3 = alpha_up.reshape(H, EK, D, D_ALPHA).transpose(0, 2, 1, 3).reshape(H, D, EK * D_ALPHA)
    bu3 = beta_up.reshape(H, EK, D, D_ALPHA).transpose(0, 2, 1, 3).reshape(H, D, EK * D_ALPHA)
    ad_bd = jnp.einsum('heak,ef->heafk', alpha_down.reshape(H, EK, D_ALPHA, DKP),
                       eyek).reshape(H, EK * D_ALPHA, EK * DKP)
    bd_bd = jnp.einsum('hea,ef->heaf', beta_down.reshape(H, EK, D_ALPHA),
                       eyek).reshape(H, EK * D_ALPHA, EK)

    positions = jnp.arange(T, dtype=jnp.float32)
    freqs = 10000.0 ** (jnp.arange(DK, dtype=jnp.float32) / DK)
    phi = positions[:, None] * freqs[None, :]
    phik = phi - 2.0 * np.pi * jax.nn.sigmoid(pope_delta)[None, :]
    cq, sq = jnp.cos(phi), jnp.sin(phi)
    ck, sk = jnp.cos(phik), jnp.sin(phik)

    n1 = T // TB1
    k1_out = pl.pallas_call(
        _k1_body,
        grid=(n1,),
        in_specs=[
            pl.BlockSpec((1, NM, TB1, D), lambda t: (0, 0, t, 0)),
            pl.BlockSpec((D, DK), lambda t: (0, 0)),
            pl.BlockSpec((D, DK), lambda t: (0, 0)),
            pl.BlockSpec((D, DV), lambda t: (0, 0)),
            pl.BlockSpec((D, H * EQ), lambda t: (0, 0)),
            pl.BlockSpec((D, H * EK), lambda t: (0, 0)),
        ],
        out_specs=[
            pl.BlockSpec((TB1, D), lambda t: (t, 0)),
            pl.BlockSpec((TB1, DK), lambda t: (t, 0)),
            pl.BlockSpec((TB1, DK), lambda t: (t, 0)),
            pl.BlockSpec((TB1, DV), lambda t: (t, 0)),
            pl.BlockSpec((TB1, H), lambda t: (t, 0)),
            pl.BlockSpec((TB1, H), lambda t: (t, 0)),
            pl.BlockSpec((TB1, H), lambda t: (t, 0)),
            pl.BlockSpec((TB1, H), lambda t: (t, 0)),
        ],
        out_shape=[
            jax.ShapeDtypeStruct((T, D), dt),
            jax.ShapeDtypeStruct((T, DK), dt),
            jax.ShapeDtypeStruct((T, DK), dt),
            jax.ShapeDtypeStruct((T, DV), dt),
            jax.ShapeDtypeStruct((T, H), jnp.int32),
            jax.ShapeDtypeStruct((T, H), dt),
            jax.ShapeDtypeStruct((T, H), jnp.int32),
            jax.ShapeDtypeStruct((T, H), dt),
        ],
    )(stream, W_q, W_k, W_v, qr_cat, kr_cat)
    x, qb, kb, vb, qidx, qgate, kidx, kgate = k1_out

    n2 = T // TB2
    qp, kp, vh = pl.pallas_call(
        _k2a_body,
        grid=(H, n2),
        in_specs=[
            pl.BlockSpec((TB2, D), lambda h, t: (t, 0)),
            pl.BlockSpec((TB2, DK), lambda h, t: (t, 0)),
            pl.BlockSpec((TB2, DK), lambda h, t: (t, 0)),
            pl.BlockSpec((TB2, DV), lambda h, t: (t, 0)),
            pl.BlockSpec((TB2, H), lambda h, t: (t, 0)),
            pl.BlockSpec((TB2, H), lambda h, t: (t, 0)),
            pl.BlockSpec((TB2, H), lambda h, t: (t, 0)),
            pl.BlockSpec((TB2, H), lambda h, t: (t, 0)),
            pl.BlockSpec((TB2, DK), lambda h, t: (t, 0)),
            pl.BlockSpec((TB2, DK), lambda h, t: (t, 0)),
            pl.BlockSpec((TB2, DK), lambda h, t: (t, 0)),
            pl.BlockSpec((TB2, DK), lambda h, t: (t, 0)),
            pl.BlockSpec((1, D, EQ * R), lambda h, t: (h, 0, 0)),
            pl.BlockSpec((1, EQ * R, EQ * DK), lambda h, t: (h, 0, 0)),
            pl.BlockSpec((1, D, EK * R), lambda h, t: (h, 0, 0)),
            pl.BlockSpec((1, EK * R, EK * DK), lambda h, t: (h, 0, 0)),
            pl.BlockSpec((1, D, EK * R), lambda h, t: (h, 0, 0)),
            pl.BlockSpec((1, EK * R, EK * DV), lambda h, t: (h, 0, 0)),
        ],
        out_specs=[
            pl.BlockSpec((1, TB2, DKP), lambda h, t: (h, t, 0)),
            pl.BlockSpec((1, TB2, DKP), lambda h, t: (h, t, 0)),
            pl.BlockSpec((1, TB2, DV), lambda h, t: (h, t, 0)),
        ],
        out_shape=[
            jax.ShapeDtypeStruct((H, T, DKP), dt),
            jax.ShapeDtypeStruct((H, T, DKP), dt),
            jax.ShapeDtypeStruct((H, T, DV), dt),
        ],
    )(x, qb, kb, vb, qidx, qgate, kidx, kgate, cq, sq, ck, sk,
      aq3, bq_bd, ak3, bk_bd, av3, bv_bd)

    n3 = T // TB3
    ax, bx = pl.pallas_call(
        _k2b_body,
        grid=(H, n3),
        in_specs=[
            pl.BlockSpec((TB3, D), lambda h, t: (t, 0)),
            pl.BlockSpec((TB3, H), lambda h, t: (t, 0)),
            pl.BlockSpec((1, D, EK * D_ALPHA), lambda h, t: (h, 0, 0)),
            pl.BlockSpec((1, EK * D_ALPHA, EK * DKP), lambda h, t: (h, 0, 0)),
            pl.BlockSpec((1, D, EK * D_ALPHA), lambda h, t: (h, 0, 0)),
            pl.BlockSpec((1, EK * D_ALPHA, EK), lambda h, t: (h, 0, 0)),
        ],
        out_specs=[
            pl.BlockSpec((1, TB3, DKP), lambda h, t: (h, t, 0)),
            pl.BlockSpec((1, TB3, H), lambda h, t: (h, t, 0)),
        ],
        out_shape=[
            jax.ShapeDtypeStruct((H, T, DKP), dt),
            jax.ShapeDtypeStruct((H, T, H), dt),
        ],
    )(x, kidx, au3, ad_bd, bu3, bd_bd)

    qx = qp.transpose(1, 2, 0)
    kx = kp.transpose(1, 2, 0)
    axt = ax.transpose(1, 2, 0)
    bxt = bx[:, :, 0].T
    vx = vh.transpose(1, 0, 2).reshape(T, H * DV)

    nsc = T // LSC
    cc = pl.pallas_call(
        _k3_body,
        grid=(nsc,),
        in_specs=[
            pl.BlockSpec((LSC, DKP, H), lambda t: (t, 0, 0)),
            pl.BlockSpec((LSC, DKP, H), lambda t: (t, 0, 0)),
            pl.BlockSpec((LSC, DKP, H), lambda t: (t, 0, 0)),
            pl.BlockSpec((LSC, H), lambda t: (t, 0)),
            pl.BlockSpec((LSC, H * DV), lambda t: (t, 0)),
        ],
        out_specs=pl.BlockSpec((LSC, H * DV), lambda t: (t, 0)),
        out_shape=jax.ShapeDtypeStruct((T, H * DV), dt),
        scratch_shapes=[pltpu.VMEM((DKP, H * DV), F32),
                        pltpu.VMEM((LSC, DKP, H * DV), F32),
                        pltpu.VMEM((LSC, DKP, H * DV), F32),
                        pltpu.VMEM((LSC, DKP, H * DV), F32),
                        pltpu.VMEM((LSC, DKP, H * DV), F32)],
    )(qx, kx, axt, bxt, vx)

    n4 = T // TB4
    y = pl.pallas_call(
        _k4_body,
        grid=(n4,),
        in_specs=[
            pl.BlockSpec((TB4, D), lambda t: (t, 0)),
            pl.BlockSpec((TB4, CONCAT), lambda t: (t, 0)),
            pl.BlockSpec((D, CONCAT), lambda t: (0, 0)),
            pl.BlockSpec((CONCAT, D), lambda t: (0, 0)),
            pl.BlockSpec((D, D_PG), lambda t: (0, 0)),
            pl.BlockSpec((D_PG, D), lambda t: (0, 0)),
        ],
        out_specs=pl.BlockSpec((TB4, D), lambda t: (t, 0)),
        out_shape=jax.ShapeDtypeStruct((T, D), dt),
    )(x, cc, W_pre, W_o, W_pg1, W_pg2)

    return jnp.broadcast_to(y[None, None, :, :], stream.shape)


# speedup vs baseline: 64.2820x; 1.0152x over previous
import numpy as np
import jax
import jax.numpy as jnp
from jax.experimental import pallas as pl
from jax.experimental.pallas import tpu as pltpu

D_IN = 1024; DK = 16; DV = 16; H = 8; EQ = 32; EK = 24
DKP = 2 * DK; R = max(DK // 4, 1); D_ALPHA = int(DK * 1.618)
CONCAT = H * DV; D_PG = max(int(CONCAT * 0.618), 1)

TB1 = 256
TB2 = 256
TB3 = 256
LSC = 128
TB4 = 256

F32 = jnp.float32


def _k1_body(stream_ref, wq_ref, wk_ref, wv_ref, qr_ref, kr_ref,
             x_ref, qb_ref, kb_ref, vb_ref,
             qidx_ref, qgate_ref, kidx_ref, kgate_ref):
    s = stream_ref[0]
    x = jnp.mean(s, axis=0)
    x_ref[...] = x
    qb_ref[...] = jnp.dot(x, wq_ref[...], preferred_element_type=F32)
    kb_ref[...] = jnp.dot(x, wk_ref[...], preferred_element_type=F32)
    vb_ref[...] = jnp.dot(x, wv_ref[...], preferred_element_type=F32)
    ql = jnp.dot(x, qr_ref[...], preferred_element_type=F32)
    kl = jnp.dot(x, kr_ref[...], preferred_element_type=F32)

    def top1(logits, E):
        idxs, gates = [], []
        for h in range(H):
            seg = logits[:, h * E:(h + 1) * E]
            m = jnp.max(seg, axis=-1, keepdims=True)
            lane = jax.lax.broadcasted_iota(jnp.int32, seg.shape, 1)
            idx = jnp.min(jnp.where(seg == m, lane, E), axis=-1, keepdims=True)
            gate = 1.0 / jnp.sum(jnp.exp(seg - m), axis=-1, keepdims=True)
            idxs.append(idx)
            gates.append(gate)
        return jnp.concatenate(idxs, axis=1), jnp.concatenate(gates, axis=1)

    qi, qg = top1(ql, EQ)
    ki, kg = top1(kl, EK)
    qidx_ref[...] = qi
    qgate_ref[...] = qg
    kidx_ref[...] = ki
    kgate_ref[...] = kg


def _col(ref, h, fill):
    lane = jax.lax.broadcasted_iota(jnp.int32, ref.shape, 1)
    return jnp.sum(jnp.where(lane == h, ref[...], fill), axis=1, keepdims=True)


def _fold_mat(EK_lanes, K):
    r = jax.lax.broadcasted_iota(jnp.int32, (EK_lanes, K), 0)
    c = jax.lax.broadcasted_iota(jnp.int32, (EK_lanes, K), 1)
    return (r % K == c).astype(F32)


def _sel(dense_all, idx, K):
    lane = jax.lax.broadcasted_iota(jnp.int32, dense_all.shape, 1)
    masked = jnp.where(lane // K == idx, dense_all, 0.0)
    return jnp.dot(masked, _fold_mat(dense_all.shape[1], K),
                   preferred_element_type=F32)


def _k2a_body(x_ref, qb_ref, kb_ref, vb_ref,
              qidx_ref, qgate_ref, kidx_ref, kgate_ref,
              cq_ref, sq_ref, ck_ref, sk_ref,
              aq_ref, bq_ref, ak_ref, bk_ref, av_ref, bv_ref,
              qp_ref, kp_ref, vh_ref):
    h = pl.program_id(0)
    x = x_ref[...]
    qi = _col(qidx_ref, h, 0)
    qg = _col(qgate_ref, h, 0.0)
    ki = _col(kidx_ref, h, 0)
    kg = _col(kgate_ref, h, 0.0)

    hq_all = jnp.dot(x, aq_ref[0], preferred_element_type=F32)
    dq_all = jnp.dot(hq_all, bq_ref[0], preferred_element_type=F32)
    dq = _sel(dq_all, qi, DK)
    qh = (qb_ref[...] + dq) * qg * (qi != 0).astype(F32)

    hk_all = jnp.dot(x, ak_ref[0], preferred_element_type=F32)
    dk_all = jnp.dot(hk_all, bk_ref[0], preferred_element_type=F32)
    kh = (kb_ref[...] + _sel(dk_all, ki, DK)) * kg

    hv_all = jnp.dot(x, av_ref[0], preferred_element_type=F32)
    dv_all = jnp.dot(hv_all, bv_ref[0], preferred_element_type=F32)
    vh_ref[0] = (vb_ref[...] + _sel(dv_all, ki, DV)) * kg

    mu_q = jax.nn.softplus(qh)
    qp_ref[0] = jnp.concatenate([mu_q * cq_ref[...], mu_q * sq_ref[...]], axis=-1)
    mu_k = jax.nn.softplus(kh)
    kp_ref[0] = jnp.concatenate([mu_k * ck_ref[...], mu_k * sk_ref[...]], axis=-1)


def _k2b_body(x_ref, kidx_ref, au_ref, ad_ref, bu_ref, bd_ref,
              ax_ref, bx_ref):
    h = pl.program_id(0)
    x = x_ref[...].astype(jnp.bfloat16)
    ki = _col(kidx_ref, h, 0)
    ha = jnp.dot(x, au_ref[0], preferred_element_type=F32)
    ha = ha * jax.nn.sigmoid(ha)
    a_all = jnp.dot(ha, ad_ref[0], preferred_element_type=F32)
    ax_ref[0] = 2.0 * jax.nn.sigmoid(_sel(a_all, ki, DKP))
    hb = jnp.dot(x, bu_ref[0], preferred_element_type=F32)
    hb = hb * jax.nn.sigmoid(hb)
    b_all = jnp.dot(hb, bd_ref[0], preferred_element_type=F32)
    lane = jax.lax.broadcasted_iota(jnp.int32, b_all.shape, 1)
    bpre = jnp.sum(jnp.where(lane == ki, b_all, 0.0), axis=1, keepdims=True)
    bx_ref[0] = jnp.broadcast_to(jax.nn.sigmoid(bpre), bx_ref.shape[1:])


def _k3_body(qx_ref, kx_ref, ax_ref, bx_ref, vx_ref, out_ref, u_ref,
             qe_s, ke_s, kbe_s, ae_s):
    @pl.when(pl.program_id(0) == 0)
    def _():
        u_ref[...] = jnp.zeros_like(u_ref)

    L = qx_ref.shape[0]
    lane = jax.lax.broadcasted_iota(jnp.int32, (H, H * DV), 1)
    row = jax.lax.broadcasted_iota(jnp.int32, (H, H * DV), 0)
    exp_m = (lane // DV == row).astype(F32)

    def expand(ref3):
        a = ref3[...].reshape(L * DKP, H)
        return jnp.dot(a, exp_m, preferred_element_type=F32).reshape(L, DKP, H * DV)

    qe_s[...] = expand(qx_ref)
    ke = expand(kx_ref)
    ke_s[...] = ke
    ae_s[...] = expand(ax_ref)
    b_exp = jnp.dot(bx_ref[...], exp_m, preferred_element_type=F32)
    kbe_s[...] = ke * b_exp[:, None, :]

    def body(i, U):
        a_t = ae_s[i]
        k_t = ke_s[i]
        kb_t = kbe_s[i]
        q_t = qe_s[i]
        v_t = vx_ref[pl.ds(i, 1), :]
        Z = a_t * U
        r = jnp.sum(k_t * Z, axis=0, keepdims=True)
        s_new = Z - kb_t * (r - v_t)
        out_ref[pl.ds(i, 1), :] = jnp.sum(q_t * s_new, axis=0, keepdims=True)
        return s_new - U

    u_ref[...] = jax.lax.fori_loop(0, L, body, u_ref[...])


def _k4_body(x_ref, cc_ref, wpre_ref, wo_ref, wpg1_ref, wpg2_ref, y_ref):
    x = x_ref[...]
    pre = jax.nn.sigmoid(jnp.dot(x, wpre_ref[...], preferred_element_type=F32))
    o = jnp.dot(cc_ref[...] * pre, wo_ref[...], preferred_element_type=F32)
    g1 = jnp.dot(x, wpg1_ref[...], preferred_element_type=F32)
    pg = jax.nn.sigmoid(jnp.dot(g1 * jax.nn.sigmoid(g1), wpg2_ref[...],
                                preferred_element_type=F32))
    y_ref[...] = o * pg


def kernel(stream, W_q, W_k, W_v, pope_delta, q_router_w, kv_router_w,
           lora_A_q, lora_B_q, lora_A_k, lora_B_k, lora_A_v, lora_B_v,
           alpha_up, alpha_down, beta_up, beta_down,
           smhc_norm_w, smhc_phi_pre, smhc_phi_post, smhc_phi_res,
           W_pre, W_o, W_pg1, W_pg2):
    Bb, NM, T, D = stream.shape
    dt = stream.dtype

    qr_cat = q_router_w.transpose(1, 0, 2).reshape(D, H * EQ)
    kr_cat = kv_router_w.transpose(1, 0, 2).reshape(D, H * EK)
    aq3 = lora_A_q.reshape(H, EQ, D, R).transpose(0, 2, 1, 3).reshape(H, D, EQ * R)
    ak3 = lora_A_k.reshape(H, EK, D, R).transpose(0, 2, 1, 3).reshape(H, D, EK * R)
    av3 = lora_A_v.reshape(H, EK, D, R).transpose(0, 2, 1, 3).reshape(H, D, EK * R)
    eyeq = jnp.eye(EQ, dtype=dt)
    eyek = jnp.eye(EK, dtype=dt)
    bq_bd = jnp.einsum('herk,ef->herfk', lora_B_q.reshape(H, EQ, R, DK),
                       eyeq).reshape(H, EQ * R, EQ * DK)
    bk_bd = jnp.einsum('herk,ef->herfk', lora_B_k.reshape(H, EK, R, DK),
                       eyek).reshape(H, EK * R, EK * DK)
    bv_bd = jnp.einsum('herk,ef->herfk', lora_B_v.reshape(H, EK, R, DV),
                       eyek).reshape(H, EK * R, EK * DV)
    au3 = alpha_up.reshape(H, EK, D, D_ALPHA).transpose(0, 2, 1, 3).reshape(
        H, D, EK * D_ALPHA).astype(jnp.bfloat16)
    bu3 = beta_up.reshape(H, EK, D, D_ALPHA).transpose(0, 2, 1, 3).reshape(
        H, D, EK * D_ALPHA).astype(jnp.bfloat16)
    ad_bd = jnp.einsum('heak,ef->heafk', alpha_down.reshape(H, EK, D_ALPHA, DKP),
                       eyek).reshape(H, EK * D_ALPHA, EK * DKP)
    bd_bd = jnp.einsum('hea,ef->heaf', beta_down.reshape(H, EK, D_ALPHA),
                       eyek).reshape(H, EK * D_ALPHA, EK)

    positions = jnp.arange(T, dtype=jnp.float32)
    freqs = 10000.0 ** (jnp.arange(DK, dtype=jnp.float32) / DK)
    phi = positions[:, None] * freqs[None, :]
    phik = phi - 2.0 * np.pi * jax.nn.sigmoid(pope_delta)[None, :]
    cq, sq = jnp.cos(phi), jnp.sin(phi)
    ck, sk = jnp.cos(phik), jnp.sin(phik)

    n1 = T // TB1
    k1_out = pl.pallas_call(
        _k1_body,
        grid=(n1,),
        in_specs=[
            pl.BlockSpec((1, NM, TB1, D), lambda t: (0, 0, t, 0)),
            pl.BlockSpec((D, DK), lambda t: (0, 0)),
            pl.BlockSpec((D, DK), lambda t: (0, 0)),
            pl.BlockSpec((D, DV), lambda t: (0, 0)),
            pl.BlockSpec((D, H * EQ), lambda t: (0, 0)),
            pl.BlockSpec((D, H * EK), lambda t: (0, 0)),
        ],
        out_specs=[
            pl.BlockSpec((TB1, D), lambda t: (t, 0)),
            pl.BlockSpec((TB1, DK), lambda t: (t, 0)),
            pl.BlockSpec((TB1, DK), lambda t: (t, 0)),
            pl.BlockSpec((TB1, DV), lambda t: (t, 0)),
            pl.BlockSpec((TB1, H), lambda t: (t, 0)),
            pl.BlockSpec((TB1, H), lambda t: (t, 0)),
            pl.BlockSpec((TB1, H), lambda t: (t, 0)),
            pl.BlockSpec((TB1, H), lambda t: (t, 0)),
        ],
        out_shape=[
            jax.ShapeDtypeStruct((T, D), dt),
            jax.ShapeDtypeStruct((T, DK), dt),
            jax.ShapeDtypeStruct((T, DK), dt),
            jax.ShapeDtypeStruct((T, DV), dt),
            jax.ShapeDtypeStruct((T, H), jnp.int32),
            jax.ShapeDtypeStruct((T, H), dt),
            jax.ShapeDtypeStruct((T, H), jnp.int32),
            jax.ShapeDtypeStruct((T, H), dt),
        ],
    )(stream, W_q, W_k, W_v, qr_cat, kr_cat)
    x, qb, kb, vb, qidx, qgate, kidx, kgate = k1_out

    n2 = T // TB2
    qp, kp, vh = pl.pallas_call(
        _k2a_body,
        grid=(H, n2),
        in_specs=[
            pl.BlockSpec((TB2, D), lambda h, t: (t, 0)),
            pl.BlockSpec((TB2, DK), lambda h, t: (t, 0)),
            pl.BlockSpec((TB2, DK), lambda h, t: (t, 0)),
            pl.BlockSpec((TB2, DV), lambda h, t: (t, 0)),
            pl.BlockSpec((TB2, H), lambda h, t: (t, 0)),
            pl.BlockSpec((TB2, H), lambda h, t: (t, 0)),
            pl.BlockSpec((TB2, H), lambda h, t: (t, 0)),
            pl.BlockSpec((TB2, H), lambda h, t: (t, 0)),
            pl.BlockSpec((TB2, DK), lambda h, t: (t, 0)),
            pl.BlockSpec((TB2, DK), lambda h, t: (t, 0)),
            pl.BlockSpec((TB2, DK), lambda h, t: (t, 0)),
            pl.BlockSpec((TB2, DK), lambda h, t: (t, 0)),
            pl.BlockSpec((1, D, EQ * R), lambda h, t: (h, 0, 0)),
            pl.BlockSpec((1, EQ * R, EQ * DK), lambda h, t: (h, 0, 0)),
            pl.BlockSpec((1, D, EK * R), lambda h, t: (h, 0, 0)),
            pl.BlockSpec((1, EK * R, EK * DK), lambda h, t: (h, 0, 0)),
            pl.BlockSpec((1, D, EK * R), lambda h, t: (h, 0, 0)),
            pl.BlockSpec((1, EK * R, EK * DV), lambda h, t: (h, 0, 0)),
        ],
        out_specs=[
            pl.BlockSpec((1, TB2, DKP), lambda h, t: (h, t, 0)),
            pl.BlockSpec((1, TB2, DKP), lambda h, t: (h, t, 0)),
            pl.BlockSpec((1, TB2, DV), lambda h, t: (h, t, 0)),
        ],
        out_shape=[
            jax.ShapeDtypeStruct((H, T, DKP), dt),
            jax.ShapeDtypeStruct((H, T, DKP), dt),
            jax.ShapeDtypeStruct((H, T, DV), dt),
        ],
    )(x, qb, kb, vb, qidx, qgate, kidx, kgate, cq, sq, ck, sk,
      aq3, bq_bd, ak3, bk_bd, av3, bv_bd)

    n3 = T // TB3
    ax, bx = pl.pallas_call(
        _k2b_body,
        grid=(H, n3),
        in_specs=[
            pl.BlockSpec((TB3, D), lambda h, t: (t, 0)),
            pl.BlockSpec((TB3, H), lambda h, t: (t, 0)),
            pl.BlockSpec((1, D, EK * D_ALPHA), lambda h, t: (h, 0, 0)),
            pl.BlockSpec((1, EK * D_ALPHA, EK * DKP), lambda h, t: (h, 0, 0)),
            pl.BlockSpec((1, D, EK * D_ALPHA), lambda h, t: (h, 0, 0)),
            pl.BlockSpec((1, EK * D_ALPHA, EK), lambda h, t: (h, 0, 0)),
        ],
        out_specs=[
            pl.BlockSpec((1, TB3, DKP), lambda h, t: (h, t, 0)),
            pl.BlockSpec((1, TB3, H), lambda h, t: (h, t, 0)),
        ],
        out_shape=[
            jax.ShapeDtypeStruct((H, T, DKP), dt),
            jax.ShapeDtypeStruct((H, T, H), dt),
        ],
    )(x, kidx, au3, ad_bd, bu3, bd_bd)

    qx = qp.transpose(1, 2, 0)
    kx = kp.transpose(1, 2, 0)
    axt = ax.transpose(1, 2, 0)
    bxt = bx[:, :, 0].T
    vx = vh.transpose(1, 0, 2).reshape(T, H * DV)

    nsc = T // LSC
    cc = pl.pallas_call(
        _k3_body,
        grid=(nsc,),
        in_specs=[
            pl.BlockSpec((LSC, DKP, H), lambda t: (t, 0, 0)),
            pl.BlockSpec((LSC, DKP, H), lambda t: (t, 0, 0)),
            pl.BlockSpec((LSC, DKP, H), lambda t: (t, 0, 0)),
            pl.BlockSpec((LSC, H), lambda t: (t, 0)),
            pl.BlockSpec((LSC, H * DV), lambda t: (t, 0)),
        ],
        out_specs=pl.BlockSpec((LSC, H * DV), lambda t: (t, 0)),
        out_shape=jax.ShapeDtypeStruct((T, H * DV), dt),
        scratch_shapes=[pltpu.VMEM((DKP, H * DV), F32),
                        pltpu.VMEM((LSC, DKP, H * DV), F32),
                        pltpu.VMEM((LSC, DKP, H * DV), F32),
                        pltpu.VMEM((LSC, DKP, H * DV), F32),
                        pltpu.VMEM((LSC, DKP, H * DV), F32)],
    )(qx, kx, axt, bxt, vx)

    n4 = T // TB4
    y = pl.pallas_call(
        _k4_body,
        grid=(n4,),
        in_specs=[
            pl.BlockSpec((TB4, D), lambda t: (t, 0)),
            pl.BlockSpec((TB4, CONCAT), lambda t: (t, 0)),
            pl.BlockSpec((D, CONCAT), lambda t: (0, 0)),
            pl.BlockSpec((CONCAT, D), lambda t: (0, 0)),
            pl.BlockSpec((D, D_PG), lambda t: (0, 0)),
            pl.BlockSpec((D_PG, D), lambda t: (0, 0)),
        ],
        out_specs=pl.BlockSpec((TB4, D), lambda t: (t, 0)),
        out_shape=jax.ShapeDtypeStruct((T, D), dt),
    )(x, cc, W_pre, W_o, W_pg1, W_pg2)

    return jnp.broadcast_to(y[None, None, :, :], stream.shape)
